# Initial kernel scaffold; baseline (speedup 1.0000x reference)
import jax
import jax.numpy as jnp
from jax.experimental import pallas as pl


def kernel(obs, is_null, W1, b1, W2, b2):
    raise NotImplementedError("write your pallas kernel here")


# speedup vs baseline: 121.2917x; 121.2917x over previous
import jax
import jax.numpy as jnp
from jax import lax
from jax.experimental import pallas as pl
from jax.experimental.pallas import tpu as pltpu

_K = 10
_CLUSTER = 0.008
_EPS = 1e-4
_C = 0.001
_SIM_MAX = 8.0

_HI = jax.lax.Precision.HIGHEST


def _body(x_ref, w1_ref, b1_ref, w2_ref, b2_ref, out_ref):
    x = x_ref[...]
    w1 = w1_ref[...]
    w2 = w2_ref[...]
    b1 = b1_ref[...]
    b2 = b2_ref[...]

    h = jnp.maximum(jnp.dot(x, w1, preferred_element_type=jnp.float32,
                            precision=_HI) + b1, 0.0)
    e = jnp.maximum(jnp.dot(h, w2, preferred_element_type=jnp.float32,
                            precision=_HI) + b2, 0.0)

    B, S = 4, 128
    row = lax.broadcasted_iota(jnp.int32, (S, S), 0)
    col = lax.broadcasted_iota(jnp.int32, (S, S), 1)
    diag = (row == col).astype(jnp.float32)

    topks = []
    bms = []
    for i in range(B):
        ei = e[i * S:(i + 1) * S, :]
        g = lax.dot_general(ei, ei, (((1,), (1,)), ((), ())),
                            preferred_element_type=jnp.float32,
                            precision=_HI)
        gd = g * diag
        sq_r = jnp.sum(gd, axis=1, keepdims=True)
        sq_c = jnp.sum(gd, axis=0, keepdims=True)
        d2 = sq_r + sq_c - 2.0 * g
        dist = jnp.sqrt(jnp.maximum(d2, 1e-24))
        dm = jnp.where(col < row, dist, jnp.inf)

        cols_k = []
        for _ in range(_K):
            mn = jnp.min(dm, axis=1, keepdims=True)
            eq = dm == mn
            first = jnp.min(jnp.where(eq, col, S + 1), axis=1,
                            keepdims=True)
            dm = jnp.where(col == first, jnp.inf, dm)
            cols_k.append(mn)
        tk = jnp.concatenate(cols_k, axis=1)
        topks.append(tk)
        bms.append(jnp.sum(tk, axis=1, keepdims=True) / _K)

    topk = jnp.concatenate(topks, axis=0)
    bm = jnp.concatenate(bms, axis=0)

    N = B * S
    fr = lax.broadcasted_iota(jnp.int32, (N, 1), 0)
    jr = lax.bitwise_and(fr, S - 1)
    valid = jr >= _K
    bm = jnp.where(valid, bm, 0.0)

    tr = lax.broadcasted_iota(jnp.int32, (N, N), 0)
    tc = lax.broadcasted_iota(jnp.int32, (N, N), 1)
    tri = (tc <= tr).astype(jnp.float32)
    cum = jnp.dot(tri, bm, preferred_element_type=jnp.float32,
                  precision=_HI)

    ir = lax.shift_right_logical(fr, 7)
    t_rank = ir * (S - _K) + jr - (_K - 1)
    count = 1e-4 + 10.0 * t_rank.astype(jnp.float32)
    rm = 10.0 * cum / count
    rm = jnp.where(valid, rm, 1.0)

    sdn = topk / (rm + 1e-11)
    sdn = jnp.maximum(sdn - _CLUSTER, 0.0)
    kern = _EPS / (sdn + _EPS)
    sim = jnp.sqrt(jnp.maximum(jnp.sum(kern, axis=1, keepdims=True), 0.0)) + _C
    r = jnp.where(sim > _SIM_MAX, 0.0, 1.0 / sim)
    out_ref[...] = jnp.where(valid, r, 0.0)


def kernel(obs, is_null, W1, b1, W2, b2):
    B, S, D = obs.shape
    x = obs.reshape(B * S, D)
    rew = pl.pallas_call(
        _body,
        out_shape=jax.ShapeDtypeStruct((B * S, 1), jnp.float32),
    )(x, W1, b1.reshape(1, -1), W2, b2.reshape(1, -1))
    rew = rew.reshape(B, S)

    null = is_null != 0
    has = jnp.any(null, axis=1)
    start = jnp.where(has, jnp.argmax(null, axis=1).astype(jnp.int32), S)
    null_cnt = jnp.sum(jnp.where(has, S - start, 0)).astype(jnp.int32)
    rew = jnp.where(jnp.arange(S)[None, :] >= start[:, None], 0.0, rew)
    er = rew.reshape(-1)
    mean = jnp.sum(er) / (B * S - null_cnt)
    return er, mean
